# baseline (speedup 1.0000x reference)
import jax
import jax.numpy as jnp
from jax.experimental import pallas as pl
from jax.experimental.pallas import tpu as pltpu

_K = 65536
_GRID = 8

_ATT_ROWS = _K // 128
_SPA_ROWS = _K * 6 // 128
_CON_ROWS = _K * 17 // 128


def _loss_kernel(attx_ref, attg_ref, spap_ref, spat_ref, conp_ref, cont_ref,
                 out_ref, acc_ref):
    i = pl.program_id(0)

    @pl.when(i == 0)
    def _init():
        acc_ref[0] = 0.0
        acc_ref[1] = 0.0
        acc_ref[2] = 0.0

    x0 = attx_ref[0]
    x1 = attx_ref[1]
    x2 = attx_ref[2]
    g = attg_ref[...]
    m = jnp.maximum(jnp.maximum(x0, x1), x2)
    s = jnp.exp(x0 - m) + jnp.exp(x1 - m) + jnp.exp(x2 - m)
    lse = m + jnp.log(s)
    xl = jnp.where(g == 0, x0, jnp.where(g == 1, x1, x2))
    ce = jnp.sum(lse - xl)

    p = jnp.clip(spap_ref[...], 1e-7, 1.0 - 1e-7)
    t = spat_ref[...]
    q = jnp.where(t == 1, p, 1.0 - p)
    bspa = -jnp.sum(jnp.log(q))

    pc = jnp.clip(conp_ref[...], 1e-7, 1.0 - 1e-7)
    tc = cont_ref[...]
    qc = jnp.where(tc == 1, pc, 1.0 - pc)
    bcon = -jnp.sum(jnp.log(qc))

    acc_ref[0] += ce
    acc_ref[1] += bspa
    acc_ref[2] += bcon

    @pl.when(i == _GRID - 1)
    def _fin():
        att = acc_ref[0] * (1.0 / _K)
        spa = acc_ref[1] * (1.0 / (_K * 6))
        con = acc_ref[2] * (1.0 / (_K * 17))
        out_ref[0] = att
        out_ref[1] = spa
        out_ref[2] = con
        out_ref[3] = att + spa + con


def kernel(att_logits, spa_probs, con_probs, att_gt, spa_gt, con_gt):
    attx = att_logits.T.reshape(3, _ATT_ROWS, 128)
    attg = att_gt.astype(jnp.int32).reshape(_ATT_ROWS, 128)
    spap = spa_probs.reshape(_SPA_ROWS, 128)
    spat = spa_gt.reshape(_SPA_ROWS, 128)
    conp = con_probs.reshape(_CON_ROWS, 128)
    cont = con_gt.reshape(_CON_ROWS, 128)

    ab = _ATT_ROWS // _GRID
    sb = _SPA_ROWS // _GRID
    cb = _CON_ROWS // _GRID

    out = pl.pallas_call(
        _loss_kernel,
        grid=(_GRID,),
        in_specs=[
            pl.BlockSpec((3, ab, 128), lambda i: (0, i, 0)),
            pl.BlockSpec((ab, 128), lambda i: (i, 0)),
            pl.BlockSpec((sb, 128), lambda i: (i, 0)),
            pl.BlockSpec((sb, 128), lambda i: (i, 0)),
            pl.BlockSpec((cb, 128), lambda i: (i, 0)),
            pl.BlockSpec((cb, 128), lambda i: (i, 0)),
        ],
        out_specs=pl.BlockSpec(memory_space=pltpu.MemorySpace.SMEM),
        out_shape=jax.ShapeDtypeStruct((4,), jnp.float32),
        scratch_shapes=[pltpu.SMEM((4,), jnp.float32)],
        compiler_params=pltpu.CompilerParams(
            dimension_semantics=("arbitrary",),
        ),
    )(attx, attg, spap, spat, conp, cont)

    att_loss = out[0]
    spa_loss = out[1]
    con_loss = out[2]
    total = out[3]
    return (att_loss, spa_loss, con_loss, total)


# speedup vs baseline: 1.0086x; 1.0086x over previous
import jax
import jax.numpy as jnp
from jax.experimental import pallas as pl
from jax.experimental.pallas import tpu as pltpu

_K = 65536
_GRID = 8

_ATT_ROWS = _K // 128
_SPA_ROWS = _K * 6 // 128
_CON_ROWS = _K * 17 // 128

_LN2 = 0.6931471805599453


def _loss_kernel(attx_ref, attg_ref, spap_ref, spat_ref, conp_ref, cont_ref,
                 out_ref, acc_ref):
    i = pl.program_id(0)

    @pl.when(i == 0)
    def _init():
        acc_ref[...] = jnp.zeros_like(acc_ref)

    x0 = attx_ref[0]
    x1 = attx_ref[1]
    x2 = attx_ref[2]
    g = attg_ref[...]
    m = jnp.maximum(jnp.maximum(x0, x1), x2)
    s = jnp.exp(x0 - m) + jnp.exp(x1 - m) + jnp.exp(x2 - m)
    lse = m + jnp.log(s)
    xl = jnp.where(g == 0, x0, jnp.where(g == 1, x1, x2))
    ce8 = jnp.sum((lse - xl).reshape(-1, 8, 128), axis=0)

    qs = jnp.abs(spap_ref[...] + spat_ref[...].astype(jnp.float32) - 1.0)
    ls = jnp.log2(jnp.maximum(qs, 1e-7))
    spa8 = jnp.sum(ls.reshape(-1, 8, 128), axis=0)

    qc = jnp.abs(conp_ref[...] + cont_ref[...].astype(jnp.float32) - 1.0)
    lc = jnp.log2(jnp.maximum(qc, 1e-7))
    con8 = jnp.sum(lc.reshape(-1, 8, 128), axis=0)

    acc_ref[0] += ce8
    acc_ref[1] += spa8
    acc_ref[2] += con8

    @pl.when(i == _GRID - 1)
    def _fin():
        att = jnp.sum(acc_ref[0]) * (1.0 / _K)
        spa = jnp.sum(acc_ref[1]) * (-_LN2 / (_K * 6))
        con = jnp.sum(acc_ref[2]) * (-_LN2 / (_K * 17))
        out_ref[0] = att
        out_ref[1] = spa
        out_ref[2] = con
        out_ref[3] = att + spa + con


def kernel(att_logits, spa_probs, con_probs, att_gt, spa_gt, con_gt):
    attx = att_logits.T.reshape(3, _ATT_ROWS, 128)
    attg = att_gt.astype(jnp.int32).reshape(_ATT_ROWS, 128)
    spap = spa_probs.reshape(_SPA_ROWS, 128)
    spat = spa_gt.reshape(_SPA_ROWS, 128)
    conp = con_probs.reshape(_CON_ROWS, 128)
    cont = con_gt.reshape(_CON_ROWS, 128)

    ab = _ATT_ROWS // _GRID
    sb = _SPA_ROWS // _GRID
    cb = _CON_ROWS // _GRID

    out = pl.pallas_call(
        _loss_kernel,
        grid=(_GRID,),
        in_specs=[
            pl.BlockSpec((3, ab, 128), lambda i: (0, i, 0)),
            pl.BlockSpec((ab, 128), lambda i: (i, 0)),
            pl.BlockSpec((sb, 128), lambda i: (i, 0)),
            pl.BlockSpec((sb, 128), lambda i: (i, 0)),
            pl.BlockSpec((cb, 128), lambda i: (i, 0)),
            pl.BlockSpec((cb, 128), lambda i: (i, 0)),
        ],
        out_specs=pl.BlockSpec(memory_space=pltpu.MemorySpace.SMEM),
        out_shape=jax.ShapeDtypeStruct((4,), jnp.float32),
        scratch_shapes=[pltpu.VMEM((3, 8, 128), jnp.float32)],
        compiler_params=pltpu.CompilerParams(
            dimension_semantics=("arbitrary",),
        ),
    )(attx, attg, spap, spat, conp, cont)

    att_loss = out[0]
    spa_loss = out[1]
    con_loss = out[2]
    total = out[3]
    return (att_loss, spa_loss, con_loss, total)


# speedup vs baseline: 14.3188x; 14.1963x over previous
import jax
import jax.numpy as jnp
from jax.experimental import pallas as pl
from jax.experimental.pallas import tpu as pltpu

_K = 65536
_GRID = 8
_B = _K // _GRID
_AB = _K // 128 // _GRID

_LN2 = 0.6931471805599453


def _loss_kernel(attx_ref, attg_ref, spap_ref, spat_ref, conp_ref, cont_ref,
                 out_ref, ce_acc, spa_acc, con_acc):
    i = pl.program_id(0)

    @pl.when(i == 0)
    def _init():
        ce_acc[...] = jnp.zeros_like(ce_acc)
        spa_acc[...] = jnp.zeros_like(spa_acc)
        con_acc[...] = jnp.zeros_like(con_acc)

    x0 = attx_ref[0]
    x1 = attx_ref[1]
    x2 = attx_ref[2]
    g = attg_ref[...]
    m = jnp.maximum(jnp.maximum(x0, x1), x2)
    s = jnp.exp(x0 - m) + jnp.exp(x1 - m) + jnp.exp(x2 - m)
    lse = m + jnp.log(s)
    xl = jnp.where(g == 0, x0, jnp.where(g == 1, x1, x2))
    ce_acc[...] += lse - xl

    qs = jnp.abs(spap_ref[...] + spat_ref[...].astype(jnp.float32) - 1.0)
    spa_acc[...] += jnp.log2(jnp.maximum(qs, 1e-7))

    qc = jnp.abs(conp_ref[...] + cont_ref[...].astype(jnp.float32) - 1.0)
    con_acc[...] += jnp.log2(jnp.maximum(qc, 1e-7))

    @pl.when(i == _GRID - 1)
    def _fin():
        att = jnp.sum(ce_acc[...]) * (1.0 / _K)
        spa = jnp.sum(spa_acc[...]) * (-_LN2 / (_K * 6))
        con = jnp.sum(con_acc[...]) * (-_LN2 / (_K * 17))
        out_ref[0] = att
        out_ref[1] = spa
        out_ref[2] = con
        out_ref[3] = att + spa + con


def kernel(att_logits, spa_probs, con_probs, att_gt, spa_gt, con_gt):
    attx = att_logits.T.reshape(3, _K // 128, 128)
    attg = att_gt.astype(jnp.int32).reshape(_K // 128, 128)
    spap = spa_probs.T
    spat = spa_gt.T
    conp = con_probs.T
    cont = con_gt.T

    out = pl.pallas_call(
        _loss_kernel,
        grid=(_GRID,),
        in_specs=[
            pl.BlockSpec((3, _AB, 128), lambda i: (0, i, 0)),
            pl.BlockSpec((_AB, 128), lambda i: (i, 0)),
            pl.BlockSpec((6, _B), lambda i: (0, i)),
            pl.BlockSpec((6, _B), lambda i: (0, i)),
            pl.BlockSpec((17, _B), lambda i: (0, i)),
            pl.BlockSpec((17, _B), lambda i: (0, i)),
        ],
        out_specs=pl.BlockSpec(memory_space=pltpu.MemorySpace.SMEM),
        out_shape=jax.ShapeDtypeStruct((4,), jnp.float32),
        scratch_shapes=[
            pltpu.VMEM((_AB, 128), jnp.float32),
            pltpu.VMEM((6, _B), jnp.float32),
            pltpu.VMEM((17, _B), jnp.float32),
        ],
        compiler_params=pltpu.CompilerParams(
            dimension_semantics=("arbitrary",),
        ),
    )(attx, attg, spap, spat, conp, cont)

    return (out[0], out[1], out[2], out[3])


# speedup vs baseline: 15.1506x; 1.0581x over previous
import jax
import jax.numpy as jnp
from jax.experimental import pallas as pl
from jax.experimental.pallas import tpu as pltpu

_K = 65536
_GRID = 2
_B = _K // _GRID
_AB = _K // 128 // _GRID

_LN2 = 0.6931471805599453


def _loss_kernel(attx_ref, attg_ref, spap_ref, spat_ref, conp_ref, cont_ref,
                 out_ref, ce_acc, spa_acc, con_acc):
    i = pl.program_id(0)

    @pl.when(i == 0)
    def _init():
        ce_acc[...] = jnp.zeros_like(ce_acc)
        spa_acc[...] = jnp.zeros_like(spa_acc)
        con_acc[...] = jnp.zeros_like(con_acc)

    x0 = attx_ref[0]
    x1 = attx_ref[1]
    x2 = attx_ref[2]
    g = attg_ref[...]
    m = jnp.maximum(jnp.maximum(x0, x1), x2)
    s = jnp.exp(x0 - m) + jnp.exp(x1 - m) + jnp.exp(x2 - m)
    lse = m + jnp.log(s)
    xl = jnp.where(g == 0, x0, jnp.where(g == 1, x1, x2))
    ce_acc[...] += lse - xl

    qs = jnp.abs(spap_ref[...] + spat_ref[...].astype(jnp.float32) - 1.0)
    spa_acc[...] += jnp.log2(jnp.maximum(qs, 1e-7))

    qc = jnp.abs(conp_ref[...] + cont_ref[...].astype(jnp.float32) - 1.0)
    con_acc[...] += jnp.log2(jnp.maximum(qc, 1e-7))

    @pl.when(i == _GRID - 1)
    def _fin():
        att = jnp.sum(ce_acc[...]) * (1.0 / _K)
        spa = jnp.sum(spa_acc[...]) * (-_LN2 / (_K * 6))
        con = jnp.sum(con_acc[...]) * (-_LN2 / (_K * 17))
        out_ref[0] = att
        out_ref[1] = spa
        out_ref[2] = con
        out_ref[3] = att + spa + con


def kernel(att_logits, spa_probs, con_probs, att_gt, spa_gt, con_gt):
    attx = att_logits.T.reshape(3, _K // 128, 128)
    attg = att_gt.astype(jnp.int32).reshape(_K // 128, 128)
    spap = spa_probs.T
    spat = spa_gt.T
    conp = con_probs.T
    cont = con_gt.T

    out = pl.pallas_call(
        _loss_kernel,
        grid=(_GRID,),
        in_specs=[
            pl.BlockSpec((3, _AB, 128), lambda i: (0, i, 0)),
            pl.BlockSpec((_AB, 128), lambda i: (i, 0)),
            pl.BlockSpec((6, _B), lambda i: (0, i)),
            pl.BlockSpec((6, _B), lambda i: (0, i)),
            pl.BlockSpec((17, _B), lambda i: (0, i)),
            pl.BlockSpec((17, _B), lambda i: (0, i)),
        ],
        out_specs=pl.BlockSpec(memory_space=pltpu.MemorySpace.SMEM),
        out_shape=jax.ShapeDtypeStruct((4,), jnp.float32),
        scratch_shapes=[
            pltpu.VMEM((_AB, 128), jnp.float32),
            pltpu.VMEM((6, _B), jnp.float32),
            pltpu.VMEM((17, _B), jnp.float32),
        ],
        compiler_params=pltpu.CompilerParams(
            dimension_semantics=("arbitrary",),
        ),
    )(attx, attg, spap, spat, conp, cont)

    return (out[0], out[1], out[2], out[3])
